# baseline (speedup 1.0000x reference)
import functools

import jax
import jax.numpy as jnp
from jax import lax
from jax.experimental import pallas as pl
from jax.experimental.pallas import tpu as pltpu
from jax.experimental.pallas import tpu_sc as plsc

M = 131072
B = 16384
D = 128
V = 1000
VPAD = 1024
BLK = 512
NEG = -1e30

_NC = 2
_NS = 16
_NW = _NC * _NS
_BPW = B // _NW
_CHUNK = 128


def _sc_gather_body(h_p, c_p, h_s, c_s, emb_t, pidx, sidx, lidx,
                    o_hpar, o_cpar, o_hprev, o_cprev, o_emb,
                    idx_v, rows_v, sem):
    wid = lax.axis_index("s") * _NC + lax.axis_index("c")
    base = wid * _BPW

    def one_gather(table, idx_hbm, out_hbm):
        for j in range(_BPW // _CHUNK):
            pltpu.sync_copy(idx_hbm.at[pl.ds(base + j * _CHUNK, _CHUNK)], idx_v)
            pltpu.async_copy(table.at[idx_v], rows_v.at[pl.ds(j * _CHUNK, _CHUNK)], sem).wait()
        pltpu.sync_copy(rows_v, out_hbm.at[pl.ds(base, _BPW)])

    one_gather(h_p, pidx, o_hpar)
    one_gather(c_p, pidx, o_cpar)
    one_gather(h_s, sidx, o_hprev)
    one_gather(c_s, sidx, o_cprev)
    one_gather(emb_t, lidx, o_emb)


def _sc_gather(h_p, c_p, h_s, c_s, emb_table, parent_idx, prev_sib_idx, labels):
    out = jax.ShapeDtypeStruct((B, D), jnp.float32)
    fn = pl.kernel(
        _sc_gather_body,
        out_type=[out] * 5,
        mesh=plsc.VectorSubcoreMesh(core_axis_name="c", subcore_axis_name="s"),
        scratch_types=[
            pltpu.VMEM((_CHUNK,), jnp.int32),
            pltpu.VMEM((_BPW, D), jnp.float32),
            pltpu.SemaphoreType.DMA,
        ],
    )
    return fn(h_p, c_p, h_s, c_s, emb_table, parent_idx, prev_sib_idx, labels)


def _tc_body(hpar, cpar, hprev, cprev, emb, ypi, ysi, lab,
             u_pt, u_st, wdw, ones16, wpred, bpred,
             wih_p, whh_p, bias_p, wih_s, whh_s, bias_s, scal,
             loss_ref, o_hp, o_cp, o_hs, o_cs):
    i = pl.program_id(0)
    n = pl.num_programs(0)

    y_p = ypi[...].astype(jnp.float32)
    y_s = ysi[...].astype(jnp.float32)
    h_par = hpar[...]
    h_prv = hprev[...]
    e16 = emb[...].astype(jnp.bfloat16)
    hpar16 = h_par.astype(jnp.bfloat16)
    hprv16 = h_prv.astype(jnp.bfloat16)

    h_pred = jnp.tanh(
        jnp.dot(hpar16, u_pt[...], preferred_element_type=jnp.float32)
        + jnp.dot(hprv16, u_st[...], preferred_element_type=jnp.float32)
    )
    h_pred16 = h_pred.astype(jnp.bfloat16)

    b_depth = scal[0]
    b_width = scal[1]

    def rowsum(x):
        parts = x[:, 0:128]
        for k in range(128, x.shape[1], 128):
            parts = parts + x[:, k:k + 128]
        return jnp.dot(parts.astype(jnp.bfloat16), ones16[...],
                       preferred_element_type=jnp.float32)[:, 0:1]

    dw = jnp.dot(h_pred16, wdw[...], preferred_element_type=jnp.float32)
    d_logit = dw[:, 0:1] + b_depth
    w_logit = dw[:, 1:2] + b_width

    def bce_row(x, y):
        return jnp.maximum(x, 0.0) + jnp.log1p(jnp.exp(-jnp.abs(x))) - x * y

    logits = (jnp.dot(h_pred16, wpred[...], preferred_element_type=jnp.float32)
              + bpred[...])
    mx = jnp.max(logits, axis=1, keepdims=True)
    lse = jnp.log(rowsum(jnp.exp(logits - mx))) + mx
    iota = lax.broadcasted_iota(jnp.int32, (BLK, VPAD), 1)
    picked = rowsum(jnp.where(iota == lab[...], logits, 0.0))
    rowtot = (lse - picked) + bce_row(d_logit, y_p) + bce_row(w_logit, y_s)
    part = jnp.sum(rowtot)

    def sig(x):
        return 0.5 * jnp.tanh(0.5 * x) + 0.5

    def lstm(x16, h16, c, w_ih, w_hh, b):
        g = (jnp.dot(x16, w_ih[...], preferred_element_type=jnp.float32)
             + jnp.dot(h16, w_hh[...], preferred_element_type=jnp.float32)
             + b[...])
        ig = sig(g[:, 0:D])
        fg = sig(g[:, D:2 * D])
        gg = jnp.tanh(g[:, 2 * D:3 * D])
        og = sig(g[:, 3 * D:4 * D])
        c2 = fg * c + ig * gg
        return og * jnp.tanh(c2), c2

    h_pn, c_pn = lstm(e16, hpar16, cpar[...], wih_p, whh_p, bias_p)
    h_sn, c_sn = lstm(e16, hprv16, cprev[...], wih_s, whh_s, bias_s)

    o_hp[...] = h_pn
    o_cp[...] = c_pn
    o_hs[...] = h_sn
    o_cs[...] = c_sn

    @pl.when(i == 0)
    def _init():
        loss_ref[0, 0] = 0.0

    loss_ref[0, 0] += part

    @pl.when(i == n - 1)
    def _fin():
        loss_ref[0, 0] = loss_ref[0, 0] * (1.0 / M)


def _tc_compute(hpar, cpar, hprev, cprev, emb, ypi, ysi, lab2,
                u_pt, u_st, wdw, ones16, wpred, bpred,
                wih_p, whh_p, bias_p, wih_s, whh_s, bias_s, scal):
    grid = B // BLK
    row = pl.BlockSpec((BLK, D), lambda i: (i, 0))
    col1 = pl.BlockSpec((BLK, 1), lambda i: (i, 0))
    full = lambda a, b: pl.BlockSpec((a, b), lambda i: (0, 0))
    out_shape = [
        jax.ShapeDtypeStruct((1, 1), jnp.float32),
        jax.ShapeDtypeStruct((B, D), jnp.float32),
        jax.ShapeDtypeStruct((B, D), jnp.float32),
        jax.ShapeDtypeStruct((B, D), jnp.float32),
        jax.ShapeDtypeStruct((B, D), jnp.float32),
    ]
    return pl.pallas_call(
        _tc_body,
        grid=grid,
        in_specs=[
            row, row, row, row, row,
            col1, col1, col1,
            full(D, D), full(D, D),
            full(D, D), full(D, D),
            full(D, VPAD), full(1, VPAD),
            full(D, 4 * D), full(D, 4 * D), full(1, 4 * D),
            full(D, 4 * D), full(D, 4 * D), full(1, 4 * D),
            pl.BlockSpec(memory_space=pltpu.SMEM),
        ],
        out_specs=[
            pl.BlockSpec(memory_space=pltpu.SMEM),
            row, row, row, row,
        ],
        out_shape=out_shape,
    )(hpar, cpar, hprev, cprev, emb, ypi, ysi, lab2,
      u_pt, u_st, wdw, ones16, wpred, bpred,
      wih_p, whh_p, bias_p, wih_s, whh_s, bias_s, scal)


def _splice_body(hpn, cpn, hsn, csn, hp_any, cp_any, hs_any, cs_any,
                 o_hp, o_cp, o_hs, o_cs):
    o_hp[...] = hpn[...]
    o_cp[...] = cpn[...]
    o_hs[...] = hsn[...]
    o_cs[...] = csn[...]


def _splice(hpn, cpn, hsn, csn, h_p, c_p, h_s, c_s):
    grid = B // BLK
    row = pl.BlockSpec((BLK, D), lambda i: (i, 0))
    anyspec = pl.BlockSpec(memory_space=pl.ANY)
    out_shape = [jax.ShapeDtypeStruct((M, D), jnp.float32)] * 4
    return pl.pallas_call(
        _splice_body,
        grid=grid,
        in_specs=[row, row, row, row, anyspec, anyspec, anyspec, anyspec],
        out_specs=[row, row, row, row],
        out_shape=out_shape,
        input_output_aliases={4: 0, 5: 1, 6: 2, 7: 3},
    )(hpn, cpn, hsn, csn, h_p, c_p, h_s, c_s)


def kernel(h_p, c_p, h_s, c_s, parent_idx, prev_sib_idx, cur_idx, labels,
           is_parent_i, has_sibling_i, U_parent, U_sibling, W_depth, b_depth,
           W_width, b_width, W_pred, b_pred, emb_table, W_ih_p, W_hh_p,
           b_ih_p, b_hh_p, W_ih_s, W_hh_s, b_ih_s, b_hh_s, off_p, off_s):
    hpar, cpar, hprev, cprev, emb = _sc_gather(
        h_p, c_p, h_s, c_s, emb_table, parent_idx, prev_sib_idx, labels)

    bf = jnp.bfloat16
    wpred = jnp.pad(W_pred, ((0, 0), (0, VPAD - V))).astype(bf)
    bpred = jnp.pad(b_pred, (0, VPAD - V), constant_values=NEG).reshape(1, VPAD)
    wdw = jnp.concatenate([W_depth, W_width, jnp.zeros((D, D - 2))], axis=1).astype(bf)
    ones16 = jnp.ones((D, D), bf)
    scal = jnp.concatenate([b_depth, b_width]).astype(jnp.float32)
    lab2 = labels.reshape(B, 1)

    loss, hpn, cpn, hsn, csn = _tc_compute(
        hpar, cpar, hprev, cprev, emb, is_parent_i, has_sibling_i, lab2,
        U_parent.T.astype(bf), U_sibling.T.astype(bf),
        wdw, ones16,
        wpred, bpred,
        W_ih_p.astype(bf), W_hh_p.astype(bf), (b_ih_p + b_hh_p).reshape(1, 4 * D),
        W_ih_s.astype(bf), W_hh_s.astype(bf), (b_ih_s + b_hh_s).reshape(1, 4 * D),
        scal)
    h_p2, c_p2, h_s2, c_s2 = _splice(hpn, cpn, hsn, csn, h_p, c_p, h_s, c_s)
    return (loss[0, 0], h_p2, c_p2, h_s2, c_s2)


# speedup vs baseline: 1.1258x; 1.1258x over previous
import functools

import jax
import jax.numpy as jnp
from jax import lax
from jax.experimental import pallas as pl
from jax.experimental.pallas import tpu as pltpu
from jax.experimental.pallas import tpu_sc as plsc

M = 131072
B = 16384
D = 128
V = 1000
VPAD = 1024
BLK = 1024
NEG = -1e30

_NC = 2
_NS = 16
_NW = _NC * _NS
_BPW = B // _NW
_CHUNK = 128


def _sc_gather_body(h_p, c_p, h_s, c_s, emb_t, pidx, sidx, lidx,
                    o_hpar, o_cpar, o_hprev, o_cprev, o_emb,
                    idx_v, rows_v, sem):
    wid = lax.axis_index("s") * _NC + lax.axis_index("c")
    base = wid * _BPW

    def one_gather(table, idx_hbm, out_hbm):
        for j in range(_BPW // _CHUNK):
            pltpu.sync_copy(idx_hbm.at[pl.ds(base + j * _CHUNK, _CHUNK)], idx_v)
            pltpu.async_copy(table.at[idx_v], rows_v.at[pl.ds(j * _CHUNK, _CHUNK)], sem).wait()
        pltpu.sync_copy(rows_v, out_hbm.at[pl.ds(base, _BPW)])

    one_gather(h_p, pidx, o_hpar)
    one_gather(c_p, pidx, o_cpar)
    one_gather(h_s, sidx, o_hprev)
    one_gather(c_s, sidx, o_cprev)
    one_gather(emb_t, lidx, o_emb)


def _sc_gather(h_p, c_p, h_s, c_s, emb_table, parent_idx, prev_sib_idx, labels):
    out = jax.ShapeDtypeStruct((B, D), jnp.float32)
    fn = pl.kernel(
        _sc_gather_body,
        out_type=[out] * 5,
        mesh=plsc.VectorSubcoreMesh(core_axis_name="c", subcore_axis_name="s"),
        scratch_types=[
            pltpu.VMEM((_CHUNK,), jnp.int32),
            pltpu.VMEM((_BPW, D), jnp.float32),
            pltpu.SemaphoreType.DMA,
        ],
    )
    return fn(h_p, c_p, h_s, c_s, emb_table, parent_idx, prev_sib_idx, labels)


def _tc_body(hpar, cpar, hprev, cprev, emb, ypi, ysi, lab,
             u_pt, u_st, wdw, ones16, wpred, bpred,
             wih_p, whh_p, bias_p, wih_s, whh_s, bias_s, scal,
             hp_any, cp_any, hs_any, cs_any,
             loss_ref, o_hp, o_cp, o_hs, o_cs):
    i = pl.program_id(0)
    n = pl.num_programs(0)

    y_p = ypi[...].astype(jnp.float32)
    y_s = ysi[...].astype(jnp.float32)
    h_par = hpar[...]
    h_prv = hprev[...]
    e16 = emb[...].astype(jnp.bfloat16)
    hpar16 = h_par.astype(jnp.bfloat16)
    hprv16 = h_prv.astype(jnp.bfloat16)

    h_pred = jnp.tanh(
        jnp.dot(hpar16, u_pt[...], preferred_element_type=jnp.float32)
        + jnp.dot(hprv16, u_st[...], preferred_element_type=jnp.float32)
    )
    h_pred16 = h_pred.astype(jnp.bfloat16)

    b_depth = scal[0]
    b_width = scal[1]

    def rowsum(x):
        parts = x[:, 0:128]
        for k in range(128, x.shape[1], 128):
            parts = parts + x[:, k:k + 128]
        return jnp.dot(parts.astype(jnp.bfloat16), ones16[...],
                       preferred_element_type=jnp.float32)[:, 0:1]

    dw = jnp.dot(h_pred16, wdw[...], preferred_element_type=jnp.float32)
    d_logit = dw[:, 0:1] + b_depth
    w_logit = dw[:, 1:2] + b_width

    def bce_row(x, y):
        return jnp.maximum(x, 0.0) + jnp.log1p(jnp.exp(-jnp.abs(x))) - x * y

    logits = (jnp.dot(h_pred16, wpred[...], preferred_element_type=jnp.float32)
              + bpred[...])
    mx = jnp.max(logits, axis=1, keepdims=True)
    lse = jnp.log(rowsum(jnp.exp(logits - mx))) + mx
    iota = lax.broadcasted_iota(jnp.int32, (BLK, VPAD), 1)
    picked = rowsum(jnp.where(iota == lab[...], logits, 0.0))
    rowtot = (lse - picked) + bce_row(d_logit, y_p) + bce_row(w_logit, y_s)
    part = jnp.sum(rowtot)

    def sig(x):
        return 0.5 * jnp.tanh(0.5 * x) + 0.5

    def lstm(x16, h16, c, w_ih, w_hh, b):
        g = (jnp.dot(x16, w_ih[...], preferred_element_type=jnp.float32)
             + jnp.dot(h16, w_hh[...], preferred_element_type=jnp.float32)
             + b[...])
        ig = sig(g[:, 0:D])
        fg = sig(g[:, D:2 * D])
        gg = jnp.tanh(g[:, 2 * D:3 * D])
        og = sig(g[:, 3 * D:4 * D])
        c2 = fg * c + ig * gg
        return og * jnp.tanh(c2), c2

    h_pn, c_pn = lstm(e16, hpar16, cpar[...], wih_p, whh_p, bias_p)
    h_sn, c_sn = lstm(e16, hprv16, cprev[...], wih_s, whh_s, bias_s)

    o_hp[...] = h_pn
    o_cp[...] = c_pn
    o_hs[...] = h_sn
    o_cs[...] = c_sn

    @pl.when(i == 0)
    def _init():
        loss_ref[0, 0] = 0.0

    loss_ref[0, 0] += part

    @pl.when(i == n - 1)
    def _fin():
        loss_ref[0, 0] = loss_ref[0, 0] * (1.0 / M)


def _tc_compute(hpar, cpar, hprev, cprev, emb, ypi, ysi, lab2,
                u_pt, u_st, wdw, ones16, wpred, bpred,
                wih_p, whh_p, bias_p, wih_s, whh_s, bias_s, scal,
                h_p, c_p, h_s, c_s):
    grid = B // BLK
    row = pl.BlockSpec((BLK, D), lambda i: (i, 0))
    col1 = pl.BlockSpec((BLK, 1), lambda i: (i, 0))
    full = lambda a, b: pl.BlockSpec((a, b), lambda i: (0, 0))
    anyspec = pl.BlockSpec(memory_space=pl.ANY)
    out_shape = [
        jax.ShapeDtypeStruct((1, 1), jnp.float32),
        jax.ShapeDtypeStruct((M, D), jnp.float32),
        jax.ShapeDtypeStruct((M, D), jnp.float32),
        jax.ShapeDtypeStruct((M, D), jnp.float32),
        jax.ShapeDtypeStruct((M, D), jnp.float32),
    ]
    return pl.pallas_call(
        _tc_body,
        grid=grid,
        in_specs=[
            row, row, row, row, row,
            col1, col1, col1,
            full(D, D), full(D, D),
            full(D, D), full(D, D),
            full(D, VPAD), full(1, VPAD),
            full(D, 4 * D), full(D, 4 * D), full(1, 4 * D),
            full(D, 4 * D), full(D, 4 * D), full(1, 4 * D),
            pl.BlockSpec(memory_space=pltpu.SMEM),
            anyspec, anyspec, anyspec, anyspec,
        ],
        out_specs=[
            pl.BlockSpec(memory_space=pltpu.SMEM),
            row, row, row, row,
        ],
        out_shape=out_shape,
        input_output_aliases={21: 1, 22: 2, 23: 3, 24: 4},
    )(hpar, cpar, hprev, cprev, emb, ypi, ysi, lab2,
      u_pt, u_st, wdw, ones16, wpred, bpred,
      wih_p, whh_p, bias_p, wih_s, whh_s, bias_s, scal,
      h_p, c_p, h_s, c_s)


def kernel(h_p, c_p, h_s, c_s, parent_idx, prev_sib_idx, cur_idx, labels,
           is_parent_i, has_sibling_i, U_parent, U_sibling, W_depth, b_depth,
           W_width, b_width, W_pred, b_pred, emb_table, W_ih_p, W_hh_p,
           b_ih_p, b_hh_p, W_ih_s, W_hh_s, b_ih_s, b_hh_s, off_p, off_s):
    hpar, cpar, hprev, cprev, emb = _sc_gather(
        h_p, c_p, h_s, c_s, emb_table, parent_idx, prev_sib_idx, labels)

    bf = jnp.bfloat16
    wpred = jnp.pad(W_pred, ((0, 0), (0, VPAD - V))).astype(bf)
    bpred = jnp.pad(b_pred, (0, VPAD - V), constant_values=NEG).reshape(1, VPAD)
    wdw = jnp.concatenate([W_depth, W_width, jnp.zeros((D, D - 2))], axis=1).astype(bf)
    ones16 = jnp.ones((D, D), bf)
    scal = jnp.concatenate([b_depth, b_width]).astype(jnp.float32)
    lab2 = labels.reshape(B, 1)

    loss, h_p2, c_p2, h_s2, c_s2 = _tc_compute(
        hpar, cpar, hprev, cprev, emb, is_parent_i, has_sibling_i, lab2,
        U_parent.T.astype(bf), U_sibling.T.astype(bf),
        wdw, ones16,
        wpred, bpred,
        W_ih_p.astype(bf), W_hh_p.astype(bf), (b_ih_p + b_hh_p).reshape(1, 4 * D),
        W_ih_s.astype(bf), W_hh_s.astype(bf), (b_ih_s + b_hh_s).reshape(1, 4 * D),
        scal,
        h_p, c_p, h_s, c_s)
    return (loss[0, 0], h_p2, c_p2, h_s2, c_s2)
